# baseline (speedup 1.0000x reference)
import jax
import jax.numpy as jnp
from jax.experimental import pallas as pl
from jax.experimental.pallas import tpu as pltpu

_B, _N, _C = 16, 5000, 200
_R = _B * _N
_BLK = 3200
_G = _R // _BLK
_DUR_THRESH = 0.05


def _post_kernel(x_ref, c_ref, lw_ref, vd_ref, off_ref,
                 scores_ref, labels_ref, t1_ref, t2_ref, mask_ref):
    x = x_ref[...]
    xi = jax.lax.bitcast_convert_type(x, jnp.int32)
    tile = jax.lax.broadcasted_iota(jnp.int32, (_BLK, _C), 1) // 8
    packed = jax.lax.bitcast_convert_type((xi | jnp.int32(31)) - tile,
                                          jnp.float32)
    xt = packed.T
    xg = xt.reshape(_C // 8, 8, _BLK)
    ma = jnp.max(xg, axis=0)
    mb = jnp.max(ma, axis=0, keepdims=True)
    mi = jax.lax.bitcast_convert_type(mb, jnp.int32)
    t_star = jnp.int32(31) - (mi & jnp.int32(31))
    rev_s = jnp.int32(7) - jax.lax.broadcasted_iota(jnp.int32, (8, _BLK), 0)
    s_hit = jnp.where(ma == mb, rev_s, jnp.int32(-1))
    s_star = jnp.int32(7) - jnp.max(s_hit, axis=0)
    labels = (t_star * 8)[0] + s_star
    val = jax.lax.bitcast_convert_type((mi & jnp.int32(-32)) | jnp.int32(16),
                                       jnp.float32)
    scores_ref[...] = jax.nn.sigmoid(val).reshape(1, 1, _BLK)
    labels_ref[...] = labels.reshape(1, 1, _BLK)

    c = c_ref[...]
    half_w = 0.5 * jnp.exp(lw_ref[...])
    off = off_ref[...]
    vd = vd_ref[...]
    t1 = jnp.clip(c - half_w + off, 0.0, vd)
    t2 = jnp.clip(c + half_w + off, 0.0, vd)
    t1_ref[...] = t1
    t2_ref[...] = t2
    mask_ref[...] = (t2 - t1 > _DUR_THRESH).astype(jnp.int8)


_row_spec = pl.BlockSpec((1, 1, _BLK), lambda b: (b, 0, 0))


@jax.jit
def _run(logits_flat, c, lw, vd_row, off_row):
    out = pl.pallas_call(
        _post_kernel,
        grid=(_G,),
        in_specs=[
            pl.BlockSpec((_BLK, _C), lambda b: (b, 0)),
            _row_spec, _row_spec, _row_spec, _row_spec,
        ],
        out_specs=[_row_spec] * 5,
        out_shape=[
            jax.ShapeDtypeStruct((_G, 1, _BLK), jnp.float32),
            jax.ShapeDtypeStruct((_G, 1, _BLK), jnp.int32),
            jax.ShapeDtypeStruct((_G, 1, _BLK), jnp.float32),
            jax.ShapeDtypeStruct((_G, 1, _BLK), jnp.float32),
            jax.ShapeDtypeStruct((_G, 1, _BLK), jnp.int8),
        ],
        compiler_params=pltpu.CompilerParams(
            dimension_semantics=("parallel",),
        ),
    )(logits_flat, c, lw, vd_row, off_row)
    return out


def kernel(pred_logits, pred_segments, video_durations, feature_durations, offsets):
    logits_flat = pred_logits.reshape(_R, _C)
    c = pred_segments[..., 0].reshape(_G, 1, _BLK)
    lw = pred_segments[..., 1].reshape(_G, 1, _BLK)
    vd_row = jnp.broadcast_to(video_durations[:, None], (_B, _N)).reshape(_G, 1, _BLK)
    off_row = jnp.broadcast_to(offsets[:, None], (_B, _N)).reshape(_G, 1, _BLK)
    scores, labels, t1, t2, mask = _run(logits_flat, c, lw, vd_row, off_row)
    segments = jnp.stack([t1.reshape(_B, _N), t2.reshape(_B, _N)], axis=-1)
    return (scores.reshape(_B, _N), labels.reshape(_B, _N), segments,
            mask.reshape(_B, _N).astype(jnp.bool_))


# speedup vs baseline: 2.2737x; 2.2737x over previous
import jax
import jax.numpy as jnp
from jax.experimental import pallas as pl
from jax.experimental.pallas import tpu as pltpu

_B, _N, _C = 16, 5000, 200
_DUR_THRESH = 0.05


def _post_kernel(x_ref, seg_ref, vd_ref, off_ref,
                 scores_ref, labels_ref, segout_ref, mask_ref):
    b = pl.program_id(0)
    x = x_ref[0]
    xi = jax.lax.bitcast_convert_type(x, jnp.int32)
    tile = jax.lax.broadcasted_iota(jnp.int32, (_N, _C), 1) // 8
    packed = jax.lax.bitcast_convert_type((xi | jnp.int32(31)) - tile,
                                          jnp.float32)
    xt = packed.T
    xg = xt.reshape(_C // 8, 8, _N)
    ma = jnp.max(xg, axis=0)
    mb = jnp.max(ma, axis=0, keepdims=True)
    mi = jax.lax.bitcast_convert_type(mb, jnp.int32)
    t_star = jnp.int32(31) - (mi & jnp.int32(31))
    rev_s = jnp.int32(7) - jax.lax.broadcasted_iota(jnp.int32, (8, _N), 0)
    s_hit = jnp.where(ma == mb, rev_s, jnp.int32(-1))
    s_star = jnp.int32(7) - jnp.max(s_hit, axis=0, keepdims=True)
    labels = t_star * 8 + s_star
    val = jax.lax.bitcast_convert_type((mi & jnp.int32(-32)) | jnp.int32(16),
                                       jnp.float32)
    scores_ref[pl.ds(b, 1), :] = jax.nn.sigmoid(val)
    labels_ref[pl.ds(b, 1), :] = labels

    cw = seg_ref[0]
    ct = cw.T
    c = ct[0:1, :]
    half_w = 0.5 * jnp.exp(ct[1:2, :])
    off = off_ref[b]
    vd = vd_ref[b]
    t1 = jnp.clip(c - half_w + off, 0.0, vd)
    t2 = jnp.clip(c + half_w + off, 0.0, vd)
    st = jnp.concatenate([t1, t2], axis=0)
    segout_ref[0] = st.T
    mask_ref[pl.ds(b, 1), :] = (t2 - t1 > _DUR_THRESH).astype(jnp.int32)


@jax.jit
def _run(pred_logits, pred_segments, video_durations, offsets):
    out = pl.pallas_call(
        _post_kernel,
        grid=(_B,),
        in_specs=[
            pl.BlockSpec((1, _N, _C), lambda b: (b, 0, 0)),
            pl.BlockSpec((1, _N, 2), lambda b: (b, 0, 0)),
            pl.BlockSpec(memory_space=pltpu.SMEM),
            pl.BlockSpec(memory_space=pltpu.SMEM),
        ],
        out_specs=[
            pl.BlockSpec((_B, _N), lambda b: (0, 0)),
            pl.BlockSpec((_B, _N), lambda b: (0, 0)),
            pl.BlockSpec((1, _N, 2), lambda b: (b, 0, 0)),
            pl.BlockSpec((_B, _N), lambda b: (0, 0)),
        ],
        out_shape=[
            jax.ShapeDtypeStruct((_B, _N), jnp.float32),
            jax.ShapeDtypeStruct((_B, _N), jnp.int32),
            jax.ShapeDtypeStruct((_B, _N, 2), jnp.float32),
            jax.ShapeDtypeStruct((_B, _N), jnp.int32),
        ],
        compiler_params=pltpu.CompilerParams(
            dimension_semantics=("arbitrary",),
        ),
    )(pred_logits, pred_segments, video_durations, offsets)
    return out


def kernel(pred_logits, pred_segments, video_durations, feature_durations, offsets):
    scores, labels, segments, mask = _run(
        pred_logits, pred_segments, video_durations, offsets)
    return scores, labels, segments, mask.astype(jnp.bool_)


# speedup vs baseline: 13.7912x; 6.0655x over previous
import jax
import jax.numpy as jnp
from jax.experimental import pallas as pl
from jax.experimental.pallas import tpu as pltpu

_B, _N, _C = 16, 5000, 200
_DUR_THRESH = 0.05


def _post_kernel(x_ref, seg_ref, vd_ref, off_ref,
                 scores_ref, labels_ref, segout_ref, mask_ref):
    b = pl.program_id(0)
    x = x_ref[0]
    xi = jax.lax.bitcast_convert_type(x, jnp.int32)
    tile = jax.lax.broadcasted_iota(jnp.int32, (_C, _N), 0) // 8
    packed = jax.lax.bitcast_convert_type((xi | jnp.int32(31)) - tile,
                                          jnp.float32)
    xg = packed.reshape(_C // 8, 8, _N)
    ma = jnp.max(xg, axis=0)
    mb = jnp.max(ma, axis=0, keepdims=True)
    mi = jax.lax.bitcast_convert_type(mb, jnp.int32)
    t_star = jnp.int32(31) - (mi & jnp.int32(31))
    rev_s = jnp.int32(7) - jax.lax.broadcasted_iota(jnp.int32, (8, _N), 0)
    s_hit = jnp.where(ma == mb, rev_s, jnp.int32(-1))
    s_star = jnp.int32(7) - jnp.max(s_hit, axis=0, keepdims=True)
    labels = t_star * 8 + s_star
    val = jax.lax.bitcast_convert_type((mi & jnp.int32(-32)) | jnp.int32(16),
                                       jnp.float32)
    scores_ref[pl.ds(b, 1), :] = jax.nn.sigmoid(val)
    labels_ref[pl.ds(b, 1), :] = labels

    sr = seg_ref[0]
    c = sr[0:1, :]
    half_w = 0.5 * jnp.exp(sr[1:2, :])
    off = off_ref[b]
    vd = vd_ref[b]
    t1 = jnp.clip(c - half_w + off, 0.0, vd)
    t2 = jnp.clip(c + half_w + off, 0.0, vd)
    segout_ref[0] = jnp.concatenate([t1, t2], axis=0)
    mask_ref[pl.ds(b, 1), :] = (t2 - t1 > _DUR_THRESH).astype(jnp.int32)


@jax.jit
def _run(logits_t, seg_t, video_durations, offsets):
    out = pl.pallas_call(
        _post_kernel,
        grid=(_B,),
        in_specs=[
            pl.BlockSpec((1, _C, _N), lambda b: (b, 0, 0)),
            pl.BlockSpec((1, 2, _N), lambda b: (b, 0, 0)),
            pl.BlockSpec(memory_space=pltpu.SMEM),
            pl.BlockSpec(memory_space=pltpu.SMEM),
        ],
        out_specs=[
            pl.BlockSpec((_B, _N), lambda b: (0, 0)),
            pl.BlockSpec((_B, _N), lambda b: (0, 0)),
            pl.BlockSpec((1, 2, _N), lambda b: (b, 0, 0)),
            pl.BlockSpec((_B, _N), lambda b: (0, 0)),
        ],
        out_shape=[
            jax.ShapeDtypeStruct((_B, _N), jnp.float32),
            jax.ShapeDtypeStruct((_B, _N), jnp.int32),
            jax.ShapeDtypeStruct((_B, 2, _N), jnp.float32),
            jax.ShapeDtypeStruct((_B, _N), jnp.int32),
        ],
        compiler_params=pltpu.CompilerParams(
            dimension_semantics=("arbitrary",),
        ),
    )(logits_t, seg_t, video_durations, offsets)
    return out


def kernel(pred_logits, pred_segments, video_durations, feature_durations, offsets):
    logits_t = jnp.transpose(pred_logits, (0, 2, 1))
    seg_t = jnp.transpose(pred_segments, (0, 2, 1))
    scores, labels, seg_out_t, mask = _run(
        logits_t, seg_t, video_durations, offsets)
    segments = jnp.transpose(seg_out_t, (0, 2, 1))
    return scores, labels, segments, mask.astype(jnp.bool_)


# speedup vs baseline: 14.3097x; 1.0376x over previous
import functools

import jax
import jax.numpy as jnp
from jax.experimental import pallas as pl
from jax.experimental.pallas import tpu as pltpu

_B, _N, _C = 16, 5000, 200
_NW = 5
_CW = _C // _NW
_DUR_THRESH = 0.05


def _post_kernel(*refs):
    x_refs = refs[:_NW]
    seg_ref, vd_ref, off_ref, scores_ref, labels_ref, segout_ref, mask_ref = refs[_NW:]
    b = pl.program_id(0)

    mas = []
    for k in range(_NW):
        x = x_refs[k][0]
        xi = jax.lax.bitcast_convert_type(x, jnp.int32)
        tile = (jax.lax.broadcasted_iota(jnp.int32, (_CW, _N), 0) // 8
                + jnp.int32(k * _CW // 8))
        packed = jax.lax.bitcast_convert_type((xi | jnp.int32(31)) - tile,
                                              jnp.float32)
        xg = packed.reshape(_CW // 8, 8, _N)
        mas.append(jnp.max(xg, axis=0))
    ma = functools.reduce(jnp.maximum, mas)
    mb = jnp.max(ma, axis=0, keepdims=True)
    mi = jax.lax.bitcast_convert_type(mb, jnp.int32)
    t_star = jnp.int32(31) - (mi & jnp.int32(31))
    rev_s = jnp.int32(7) - jax.lax.broadcasted_iota(jnp.int32, (8, _N), 0)
    s_hit = jnp.where(ma == mb, rev_s, jnp.int32(-1))
    s_star = jnp.int32(7) - jnp.max(s_hit, axis=0, keepdims=True)
    labels = t_star * 8 + s_star
    val = jax.lax.bitcast_convert_type((mi & jnp.int32(-32)) | jnp.int32(16),
                                       jnp.float32)
    scores_ref[pl.ds(b, 1), :] = jax.nn.sigmoid(val)
    labels_ref[pl.ds(b, 1), :] = labels

    sr = seg_ref[0]
    c = sr[0:1, :]
    half_w = 0.5 * jnp.exp(sr[1:2, :])
    off = off_ref[b]
    vd = vd_ref[b]
    t1 = jnp.clip(c - half_w + off, 0.0, vd)
    t2 = jnp.clip(c + half_w + off, 0.0, vd)
    segout_ref[0] = jnp.concatenate([t1, t2], axis=0)
    mask_ref[pl.ds(b, 1), :] = (t2 - t1 > _DUR_THRESH).astype(jnp.int32)


def _win_spec(k):
    return pl.BlockSpec((1, _CW, _N), lambda b, k=k: (b, k, 0))


@jax.jit
def _run(logits_t, seg_t, video_durations, offsets):
    out = pl.pallas_call(
        _post_kernel,
        grid=(_B,),
        in_specs=[_win_spec(k) for k in range(_NW)] + [
            pl.BlockSpec((1, 2, _N), lambda b: (b, 0, 0)),
            pl.BlockSpec(memory_space=pltpu.SMEM),
            pl.BlockSpec(memory_space=pltpu.SMEM),
        ],
        out_specs=[
            pl.BlockSpec((_B, _N), lambda b: (0, 0)),
            pl.BlockSpec((_B, _N), lambda b: (0, 0)),
            pl.BlockSpec((1, 2, _N), lambda b: (b, 0, 0)),
            pl.BlockSpec((_B, _N), lambda b: (0, 0)),
        ],
        out_shape=[
            jax.ShapeDtypeStruct((_B, _N), jnp.float32),
            jax.ShapeDtypeStruct((_B, _N), jnp.int32),
            jax.ShapeDtypeStruct((_B, 2, _N), jnp.float32),
            jax.ShapeDtypeStruct((_B, _N), jnp.int32),
        ],
        compiler_params=pltpu.CompilerParams(
            dimension_semantics=("arbitrary",),
        ),
    )(*([logits_t] * _NW), seg_t, video_durations, offsets)
    return out


def kernel(pred_logits, pred_segments, video_durations, feature_durations, offsets):
    logits_t = jnp.transpose(pred_logits, (0, 2, 1))
    seg_t = jnp.transpose(pred_segments, (0, 2, 1))
    scores, labels, seg_out_t, mask = _run(
        logits_t, seg_t, video_durations, offsets)
    segments = jnp.transpose(seg_out_t, (0, 2, 1))
    return scores, labels, segments, mask.astype(jnp.bool_)


# speedup vs baseline: 14.3184x; 1.0006x over previous
import functools

import jax
import jax.numpy as jnp
from jax.experimental import pallas as pl
from jax.experimental.pallas import tpu as pltpu

_B, _N, _C = 16, 5000, 200
_NW = 5
_CW = _C // _NW
_DUR_THRESH = 0.05


def _post_kernel(*refs):
    x_refs = refs[:_NW]
    seg_ref, vd_ref, off_ref, scores_ref, labels_ref, segout_ref, mask_ref = refs[_NW:]
    b = pl.program_id(0)

    mas = []
    for k in range(_NW):
        x = x_refs[k][0]
        xi = jax.lax.bitcast_convert_type(x, jnp.int32)
        tile = (jax.lax.broadcasted_iota(jnp.int32, (_CW, _N), 0) // 8
                + jnp.int32(k * _CW // 8))
        packed = jax.lax.bitcast_convert_type((xi | jnp.int32(31)) - tile,
                                              jnp.float32)
        xg = packed.reshape(_CW // 8, 8, _N)
        mas.append(jnp.max(xg, axis=0))
    ma = functools.reduce(jnp.maximum, mas)
    mb = jnp.max(ma, axis=0, keepdims=True)
    mi = jax.lax.bitcast_convert_type(mb, jnp.int32)
    t_star = jnp.int32(31) - (mi & jnp.int32(31))
    rev_s = jnp.int32(7) - jax.lax.broadcasted_iota(jnp.int32, (8, _N), 0)
    s_hit = jnp.where(ma == mb, rev_s, jnp.int32(-1))
    s_star = jnp.int32(7) - jnp.max(s_hit, axis=0, keepdims=True)
    labels = t_star * 8 + s_star
    val = jax.lax.bitcast_convert_type((mi & jnp.int32(-32)) | jnp.int32(16),
                                       jnp.float32)
    scores_ref[pl.ds(b, 1), :] = jax.nn.sigmoid(val)
    labels_ref[pl.ds(b, 1), :] = labels

    sr = seg_ref[0]
    c = sr[0:1, :]
    half_w = 0.5 * jnp.exp(sr[1:2, :])
    off = off_ref[b]
    vd = vd_ref[b]
    t1 = jnp.clip(c - half_w + off, 0.0, vd)
    t2 = jnp.clip(c + half_w + off, 0.0, vd)
    segout_ref[0] = jnp.concatenate([t1, t2], axis=0)
    mask_ref[pl.ds(b, 1), :] = t2 - t1 > _DUR_THRESH


def _win_spec(k):
    return pl.BlockSpec((1, _CW, _N), lambda b, k=k: (b, k, 0))


@jax.jit
def _run(logits_t, seg_t, video_durations, offsets):
    out = pl.pallas_call(
        _post_kernel,
        grid=(_B,),
        in_specs=[_win_spec(k) for k in range(_NW)] + [
            pl.BlockSpec((1, 2, _N), lambda b: (b, 0, 0)),
            pl.BlockSpec(memory_space=pltpu.SMEM),
            pl.BlockSpec(memory_space=pltpu.SMEM),
        ],
        out_specs=[
            pl.BlockSpec((_B, _N), lambda b: (0, 0)),
            pl.BlockSpec((_B, _N), lambda b: (0, 0)),
            pl.BlockSpec((1, 2, _N), lambda b: (b, 0, 0)),
            pl.BlockSpec((_B, _N), lambda b: (0, 0)),
        ],
        out_shape=[
            jax.ShapeDtypeStruct((_B, _N), jnp.float32),
            jax.ShapeDtypeStruct((_B, _N), jnp.int32),
            jax.ShapeDtypeStruct((_B, 2, _N), jnp.float32),
            jax.ShapeDtypeStruct((_B, _N), jnp.bool_),
        ],
        compiler_params=pltpu.CompilerParams(
            dimension_semantics=("arbitrary",),
        ),
    )(*([logits_t] * _NW), seg_t, video_durations, offsets)
    return out


def kernel(pred_logits, pred_segments, video_durations, feature_durations, offsets):
    logits_t = jnp.transpose(pred_logits, (0, 2, 1))
    seg_t = jnp.transpose(pred_segments, (0, 2, 1))
    scores, labels, seg_out_t, mask = _run(
        logits_t, seg_t, video_durations, offsets)
    segments = jnp.transpose(seg_out_t, (0, 2, 1))
    return scores, labels, segments, mask


# speedup vs baseline: 16.1428x; 1.1274x over previous
import functools

import jax
import jax.numpy as jnp
from jax.experimental import pallas as pl
from jax.experimental.pallas import tpu as pltpu

_B, _N, _C = 16, 5000, 200
_NW = 5
_CW = _C // _NW
_DUR_THRESH = 0.05


_BPS = 2


def _post_kernel(*refs):
    x_refs = refs[:_NW]
    seg_ref, vd_ref, off_ref, scores_ref, labels_ref, segout_ref, mask_ref = refs[_NW:]
    g = pl.program_id(0)
    for j in range(_BPS):
        _one_batch(j, g * _BPS + j, x_refs, seg_ref, vd_ref, off_ref,
                   scores_ref, labels_ref, segout_ref, mask_ref)


def _one_batch(j, b, x_refs, seg_ref, vd_ref, off_ref,
               scores_ref, labels_ref, segout_ref, mask_ref):
    mas = []
    for k in range(_NW):
        x = x_refs[k][j]
        xi = jax.lax.bitcast_convert_type(x, jnp.int32)
        tile = (jax.lax.broadcasted_iota(jnp.int32, (_CW, _N), 0) // 8
                + jnp.int32(k * _CW // 8))
        packed = jax.lax.bitcast_convert_type((xi | jnp.int32(31)) - tile,
                                              jnp.float32)
        xg = packed.reshape(_CW // 8, 8, _N)
        mas.append(jnp.max(xg, axis=0))
    ma = functools.reduce(jnp.maximum, mas)
    mb = jnp.max(ma, axis=0, keepdims=True)
    mi = jax.lax.bitcast_convert_type(mb, jnp.int32)
    t_star = jnp.int32(31) - (mi & jnp.int32(31))
    rev_s = jnp.int32(7) - jax.lax.broadcasted_iota(jnp.int32, (8, _N), 0)
    s_hit = jnp.where(ma == mb, rev_s, jnp.int32(-1))
    s_star = jnp.int32(7) - jnp.max(s_hit, axis=0, keepdims=True)
    labels = t_star * 8 + s_star
    val = jax.lax.bitcast_convert_type((mi & jnp.int32(-32)) | jnp.int32(16),
                                       jnp.float32)
    scores_ref[pl.ds(b, 1), :] = jax.nn.sigmoid(val)
    labels_ref[pl.ds(b, 1), :] = labels

    sr = seg_ref[j]
    c = sr[0:1, :]
    half_w = 0.5 * jnp.exp(sr[1:2, :])
    off = off_ref[b]
    vd = vd_ref[b]
    t1 = jnp.clip(c - half_w + off, 0.0, vd)
    t2 = jnp.clip(c + half_w + off, 0.0, vd)
    segout_ref[j] = jnp.concatenate([t1, t2], axis=0)
    mask_ref[pl.ds(b, 1), :] = t2 - t1 > _DUR_THRESH


def _win_spec(k):
    return pl.BlockSpec((_BPS, _CW, _N), lambda g, k=k: (g, k, 0))


@jax.jit
def _run(logits_t, seg_t, video_durations, offsets):
    out = pl.pallas_call(
        _post_kernel,
        grid=(_B // _BPS,),
        in_specs=[_win_spec(k) for k in range(_NW)] + [
            pl.BlockSpec((_BPS, 2, _N), lambda g: (g, 0, 0)),
            pl.BlockSpec(memory_space=pltpu.SMEM),
            pl.BlockSpec(memory_space=pltpu.SMEM),
        ],
        out_specs=[
            pl.BlockSpec((_B, _N), lambda g: (0, 0)),
            pl.BlockSpec((_B, _N), lambda g: (0, 0)),
            pl.BlockSpec((_BPS, 2, _N), lambda g: (g, 0, 0)),
            pl.BlockSpec((_B, _N), lambda g: (0, 0)),
        ],
        out_shape=[
            jax.ShapeDtypeStruct((_B, _N), jnp.float32),
            jax.ShapeDtypeStruct((_B, _N), jnp.int32),
            jax.ShapeDtypeStruct((_B, 2, _N), jnp.float32),
            jax.ShapeDtypeStruct((_B, _N), jnp.bool_),
        ],
        compiler_params=pltpu.CompilerParams(
            dimension_semantics=("arbitrary",),
        ),
    )(*([logits_t] * _NW), seg_t, video_durations, offsets)
    return out


def kernel(pred_logits, pred_segments, video_durations, feature_durations, offsets):
    logits_t = jnp.transpose(pred_logits, (0, 2, 1))
    seg_t = jnp.transpose(pred_segments, (0, 2, 1))
    scores, labels, seg_out_t, mask = _run(
        logits_t, seg_t, video_durations, offsets)
    segments = jnp.transpose(seg_out_t, (0, 2, 1))
    return scores, labels, segments, mask
